# baseline (speedup 1.0000x reference)
import functools

import jax
import jax.numpy as jnp
from jax import lax
from jax.experimental import pallas as pl
from jax.experimental.pallas import tpu as pltpu, tpu_sc as plsc

E = 3_200_000
IN_DIM = 64
OUT_DIM = 16
NC = 2
NS = 16
NW = NC * NS
CTOT = E // 128
CE = 23
CHUNK_E = CE * 128
NCH = 34
N_PAIRS = NCH // 2


def _make_gather():
    mesh = plsc.VectorSubcoreMesh(core_axis_name="c", subcore_axis_name="s")

    @functools.partial(
        pl.kernel,
        mesh=mesh,
        out_type=(
            jax.ShapeDtypeStruct((2, CTOT, 8, 128), jnp.float32),
            jax.ShapeDtypeStruct((CTOT, 2, 128), jnp.int32),
        ),
        scratch_types=[
            pltpu.VMEM((IN_DIM * OUT_DIM * 16,), jnp.float32),
            pltpu.VMEM((CHUNK_E,), jnp.int32),
            pltpu.VMEM((CHUNK_E,), jnp.int32),
            pltpu.VMEM((2, CE, 8, 128), jnp.float32),
            pltpu.VMEM((2, CE, 8, 128), jnp.float32),
            pltpu.VMEM((CE, 2, 128), jnp.int32),
            pltpu.VMEM((CE, 2, 128), jnp.int32),
            pltpu.SemaphoreType.DMA,
            pltpu.SemaphoreType.DMA,
            pltpu.SemaphoreType.DMA,
            pltpu.SemaphoreType.DMA,
            pltpu.SemaphoreType.DMA,
            pltpu.SemaphoreType.DMA,
            pltpu.SemaphoreType.DMA,
            pltpu.SemaphoreType.DMA,
        ],
        compiler_params=pltpu.CompilerParams(use_tc_tiling_on_sc=False,
                                             needs_layout_passes=False),
    )
    def gather_kernel(table_hbm, idx_hbm, spdidx_hbm, out_hbm, outidx_hbm,
                      tab_v, idx0, idx1, blk0, blk1, pbuf0, pbuf1,
                      isem0, isem1, osem0, osem1,
                      plsem0, plsem1, pssem0, pssem1):
        wid = lax.axis_index("s") * NC + lax.axis_index("c")
        cstart = wid * 781 + lax.min(wid, 8)
        ccnt = lax.select(wid < 8, 782, 781)

        pltpu.sync_copy(table_hbm, tab_v)

        iota = lax.iota(jnp.int32, 16)
        jvecs = [iota + (j * 16) for j in range(OUT_DIM)]

        def chunk_c(t):
            return cstart + lax.min(t * CE, ccnt - CE)

        def expand(idx_ref, blk_ref):
            @plsc.parallel_loop(0, CE, 1, unroll=4)
            def _(cc):
                for g in range(8):
                    iv = idx_ref[pl.ds(cc * 128 + g * 16, 16)]
                    base = iv * (OUT_DIM * 16)
                    for j in range(OUT_DIM):
                        col = plsc.load_gather(tab_v, [base + jvecs[j]])
                        blk_ref[j // 8, cc, j % 8, pl.ds(g * 16, 16)] = col

        pltpu.async_copy(idx_hbm.at[pl.ds(chunk_c(0) * 128, CHUNK_E)],
                         idx0, isem0)
        pltpu.async_copy(idx_hbm.at[pl.ds(chunk_c(1) * 128, CHUNK_E)],
                         idx1, isem1)

        def pair_body(p, carry):
            for b, (idx_v, blk_v, pbuf, isem, osem, plsem, pssem) in enumerate(
                    ((idx0, blk0, pbuf0, isem0, osem0, plsem0, pssem0),
                     (idx1, blk1, pbuf1, isem1, osem1, plsem1, pssem1))):
                t = 2 * p + b
                c = chunk_c(t)
                pltpu.make_async_copy(
                    idx_hbm.at[pl.ds(c * 128, CHUNK_E)], idx_v, isem).wait()

                @pl.when(p >= 1)
                def _():
                    pltpu.make_async_copy(
                        blk_v, out_hbm.at[:, pl.ds(c, CE)], osem).wait()
                    pltpu.make_async_copy(
                        pbuf, outidx_hbm.at[pl.ds(c, CE)], pssem).wait()

                pltpu.async_copy(spdidx_hbm.at[pl.ds(c, CE)], pbuf, plsem)

                expand(idx_v, blk_v)
                pltpu.async_copy(blk_v, out_hbm.at[:, pl.ds(c, CE)], osem)
                pltpu.make_async_copy(
                    spdidx_hbm.at[pl.ds(c, CE)], pbuf, plsem).wait()
                pltpu.async_copy(pbuf, outidx_hbm.at[pl.ds(c, CE)], pssem)

                @pl.when(t + 2 < NCH)
                def _():
                    pltpu.async_copy(
                        idx_hbm.at[pl.ds(chunk_c(t + 2) * 128, CHUNK_E)],
                        idx_v, isem)
            return carry

        lax.fori_loop(0, N_PAIRS, pair_body, 0)

        pltpu.make_async_copy(blk0, out_hbm.at[:, pl.ds(0, CE)], osem0).wait()
        pltpu.make_async_copy(blk1, out_hbm.at[:, pl.ds(0, CE)], osem1).wait()
        pltpu.make_async_copy(pbuf0, outidx_hbm.at[pl.ds(0, CE)], pssem0).wait()
        pltpu.make_async_copy(pbuf1, outidx_hbm.at[pl.ds(0, CE)], pssem1).wait()

    return gather_kernel


_gather = _make_gather()


def kernel(spd_index, spd_val, edge_index, spd_emb_weight):
    px = spd_index.T.reshape(CTOT, 128, 2).transpose(0, 2, 1)
    tab_r = jnp.broadcast_to(spd_emb_weight[:, :, None],
                             (IN_DIM, OUT_DIM, 16)).reshape(-1)
    v4, o4 = _gather(tab_r, spd_val, px)
    out_val = v4.transpose(1, 3, 0, 2).reshape(E, OUT_DIM)
    out_idx = o4.transpose(0, 2, 1).reshape(E, 2).T
    return (out_idx, out_val)


# speedup vs baseline: 1.0359x; 1.0359x over previous
import functools

import jax
import jax.numpy as jnp
from jax import lax
from jax.experimental import pallas as pl
from jax.experimental.pallas import tpu as pltpu, tpu_sc as plsc

E = 3_200_000
IN_DIM = 64
OUT_DIM = 16
NC = 2
NS = 16
NW = NC * NS
CTOT = E // 128
CE = 21
CHUNK_E = CE * 128
NCH = 38
N_PAIRS = NCH // 2


def _make_gather():
    mesh = plsc.VectorSubcoreMesh(core_axis_name="c", subcore_axis_name="s")

    @functools.partial(
        pl.kernel,
        mesh=mesh,
        out_type=(
            jax.ShapeDtypeStruct((2, CTOT, 8, 128), jnp.float32),
            jax.ShapeDtypeStruct((CTOT, 2, 128), jnp.int32),
        ),
        scratch_types=[
            pltpu.VMEM((IN_DIM * OUT_DIM * 16,), jnp.float32),
            pltpu.VMEM((CHUNK_E,), jnp.int32),
            pltpu.VMEM((CHUNK_E,), jnp.int32),
            pltpu.VMEM((2, CE, 8, 128), jnp.float32),
            pltpu.VMEM((2, CE, 8, 128), jnp.float32),
            pltpu.VMEM((CE, 2, 128), jnp.int32),
            pltpu.VMEM((CE, 2, 128), jnp.int32),
            pltpu.SemaphoreType.DMA,
            pltpu.SemaphoreType.DMA,
            pltpu.SemaphoreType.DMA,
            pltpu.SemaphoreType.DMA,
            pltpu.SemaphoreType.DMA,
            pltpu.SemaphoreType.DMA,
            pltpu.SemaphoreType.DMA,
            pltpu.SemaphoreType.DMA,
        ],
        compiler_params=pltpu.CompilerParams(use_tc_tiling_on_sc=False,
                                             needs_layout_passes=False),
    )
    def gather_kernel(table_hbm, idx_hbm, spdidx_hbm, out_hbm, outidx_hbm,
                      tab_v, idx0, idx1, blk0, blk1, pbuf0, pbuf1,
                      isem0, isem1, osem0, osem1,
                      plsem0, plsem1, pssem0, pssem1):
        wid = lax.axis_index("s") * NC + lax.axis_index("c")
        cstart = wid * 781 + lax.min(wid, 8)
        ccnt = lax.select(wid < 8, 782, 781)

        pltpu.sync_copy(table_hbm, tab_v)

        iota = lax.iota(jnp.int32, 16)
        jvecs = [iota + (j * 16) for j in range(OUT_DIM)]

        def chunk_c(t):
            return cstart + lax.min(t * CE, ccnt - CE)

        def expand(idx_ref, blk_ref):
            @plsc.parallel_loop(0, CE, 1, unroll=4)
            def _(cc):
                for g in range(8):
                    iv = idx_ref[pl.ds(cc * 128 + g * 16, 16)]
                    base = iv * (OUT_DIM * 16)
                    for j in range(OUT_DIM):
                        col = plsc.load_gather(tab_v, [base + jvecs[j]])
                        blk_ref[j // 8, cc, j % 8, pl.ds(g * 16, 16)] = col

        pltpu.async_copy(idx_hbm.at[pl.ds(chunk_c(0) * 128, CHUNK_E)],
                         idx0, isem0)
        pltpu.async_copy(idx_hbm.at[pl.ds(chunk_c(1) * 128, CHUNK_E)],
                         idx1, isem1)

        def pair_body(p, carry):
            for b, (idx_v, blk_v, pbuf, isem, osem, plsem, pssem) in enumerate(
                    ((idx0, blk0, pbuf0, isem0, osem0, plsem0, pssem0),
                     (idx1, blk1, pbuf1, isem1, osem1, plsem1, pssem1))):
                t = 2 * p + b
                c = chunk_c(t)
                pltpu.make_async_copy(
                    idx_hbm.at[pl.ds(c * 128, CHUNK_E)], idx_v, isem).wait()

                @pl.when(p >= 1)
                def _():
                    pltpu.make_async_copy(
                        blk_v, out_hbm.at[:, pl.ds(c, CE)], osem).wait()
                    pltpu.make_async_copy(
                        pbuf, outidx_hbm.at[pl.ds(c, CE)], pssem).wait()

                pltpu.async_copy(spdidx_hbm.at[pl.ds(c, CE)], pbuf, plsem)

                expand(idx_v, blk_v)
                pltpu.async_copy(blk_v, out_hbm.at[:, pl.ds(c, CE)], osem)
                pltpu.make_async_copy(
                    spdidx_hbm.at[pl.ds(c, CE)], pbuf, plsem).wait()
                pltpu.async_copy(pbuf, outidx_hbm.at[pl.ds(c, CE)], pssem)

                @pl.when(t + 2 < NCH)
                def _():
                    pltpu.async_copy(
                        idx_hbm.at[pl.ds(chunk_c(t + 2) * 128, CHUNK_E)],
                        idx_v, isem)
            return carry

        lax.fori_loop(0, N_PAIRS, pair_body, 0)

        pltpu.make_async_copy(blk0, out_hbm.at[:, pl.ds(0, CE)], osem0).wait()
        pltpu.make_async_copy(blk1, out_hbm.at[:, pl.ds(0, CE)], osem1).wait()
        pltpu.make_async_copy(pbuf0, outidx_hbm.at[pl.ds(0, CE)], pssem0).wait()
        pltpu.make_async_copy(pbuf1, outidx_hbm.at[pl.ds(0, CE)], pssem1).wait()

    return gather_kernel


_gather = _make_gather()


def kernel(spd_index, spd_val, edge_index, spd_emb_weight):
    px = spd_index.T.reshape(CTOT, 128, 2).transpose(0, 2, 1)
    tab_r = jnp.broadcast_to(spd_emb_weight[:, :, None],
                             (IN_DIM, OUT_DIM, 16)).reshape(-1)
    v4, o4 = _gather(tab_r, spd_val, px)
    out_val = v4.transpose(1, 3, 0, 2).reshape(E, OUT_DIM)
    out_idx = o4.transpose(0, 2, 1).reshape(E, 2).T
    return (out_idx, out_val)


# speedup vs baseline: 1.2276x; 1.1850x over previous
import functools

import jax
import jax.numpy as jnp
from jax import lax
from jax.experimental import pallas as pl
from jax.experimental.pallas import tpu as pltpu, tpu_sc as plsc

E = 3_200_000
IN_DIM = 64
OUT_DIM = 16
NC = 2
NS = 16
NW = NC * NS
CTOT = E // 128
CE = 21
CHUNK_E = CE * 128
NCH = 38
N_PAIRS = NCH // 2


def _make_gather():
    mesh = plsc.VectorSubcoreMesh(core_axis_name="c", subcore_axis_name="s")

    @functools.partial(
        pl.kernel,
        mesh=mesh,
        out_type=(
            jax.ShapeDtypeStruct((2, CTOT, 8, 128), jnp.float32),
            jax.ShapeDtypeStruct((CTOT, 2, 128), jnp.int32),
        ),
        scratch_types=[
            pltpu.VMEM((IN_DIM * OUT_DIM * 16,), jnp.float32),
            pltpu.VMEM((CHUNK_E,), jnp.int32),
            pltpu.VMEM((CHUNK_E,), jnp.int32),
            pltpu.VMEM((2, CE, 8, 128), jnp.float32),
            pltpu.VMEM((2, CE, 8, 128), jnp.float32),
            pltpu.VMEM((CE, 2, 128), jnp.int32),
            pltpu.VMEM((CE, 2, 128), jnp.int32),
            pltpu.SemaphoreType.DMA,
            pltpu.SemaphoreType.DMA,
            pltpu.SemaphoreType.DMA,
            pltpu.SemaphoreType.DMA,
            pltpu.SemaphoreType.DMA,
            pltpu.SemaphoreType.DMA,
            pltpu.SemaphoreType.DMA,
            pltpu.SemaphoreType.DMA,
        ],
        compiler_params=pltpu.CompilerParams(use_tc_tiling_on_sc=False,
                                             needs_layout_passes=False),
    )
    def gather_kernel(table_hbm, idx_hbm, spdidx_hbm, out_hbm, outidx_hbm,
                      tab_v, idx0, idx1, blk0, blk1, pbuf0, pbuf1,
                      isem0, isem1, osem0, osem1,
                      plsem0, plsem1, pssem0, pssem1):
        wid = lax.axis_index("s") * NC + lax.axis_index("c")
        cstart = wid * 781 + lax.min(wid, 8)
        ccnt = lax.select(wid < 8, 782, 781)

        pltpu.sync_copy(table_hbm, tab_v)

        iota = lax.iota(jnp.int32, 16)
        jvecs = [iota + (j * 16) for j in range(OUT_DIM)]

        def chunk_c(t):
            return cstart + lax.min(t * CE, ccnt - CE)

        def expand(idx_ref, blk_ref):
            @plsc.parallel_loop(0, CE, 1, unroll=2)
            def _(cc):
                for g in range(8):
                    iv = idx_ref[pl.ds(cc * 128 + g * 16, 16)]
                    base = iv * (OUT_DIM * 16)
                    for j in range(OUT_DIM):
                        col = plsc.load_gather(tab_v, [base + jvecs[j]])
                        blk_ref[j // 8, cc, j % 8, pl.ds(g * 16, 16)] = col

        pltpu.async_copy(idx_hbm.at[pl.ds(chunk_c(0) * 128, CHUNK_E)],
                         idx0, isem0)
        pltpu.async_copy(idx_hbm.at[pl.ds(chunk_c(1) * 128, CHUNK_E)],
                         idx1, isem1)

        def pair_body(p, carry):
            for b, (idx_v, blk_v, pbuf, isem, osem, plsem, pssem) in enumerate(
                    ((idx0, blk0, pbuf0, isem0, osem0, plsem0, pssem0),
                     (idx1, blk1, pbuf1, isem1, osem1, plsem1, pssem1))):
                t = 2 * p + b
                c = chunk_c(t)
                pltpu.make_async_copy(
                    idx_hbm.at[pl.ds(c * 128, CHUNK_E)], idx_v, isem).wait()

                @pl.when(p >= 1)
                def _():
                    pltpu.make_async_copy(
                        blk_v, out_hbm.at[:, pl.ds(c, CE)], osem).wait()
                    pltpu.make_async_copy(
                        pbuf, outidx_hbm.at[pl.ds(c, CE)], pssem).wait()

                pltpu.async_copy(spdidx_hbm.at[pl.ds(c, CE)], pbuf, plsem)

                expand(idx_v, blk_v)
                pltpu.async_copy(blk_v, out_hbm.at[:, pl.ds(c, CE)], osem)
                pltpu.make_async_copy(
                    spdidx_hbm.at[pl.ds(c, CE)], pbuf, plsem).wait()
                pltpu.async_copy(pbuf, outidx_hbm.at[pl.ds(c, CE)], pssem)

                @pl.when(t + 2 < NCH)
                def _():
                    pltpu.async_copy(
                        idx_hbm.at[pl.ds(chunk_c(t + 2) * 128, CHUNK_E)],
                        idx_v, isem)
            return carry

        lax.fori_loop(0, N_PAIRS, pair_body, 0)

        pltpu.make_async_copy(blk0, out_hbm.at[:, pl.ds(0, CE)], osem0).wait()
        pltpu.make_async_copy(blk1, out_hbm.at[:, pl.ds(0, CE)], osem1).wait()
        pltpu.make_async_copy(pbuf0, outidx_hbm.at[pl.ds(0, CE)], pssem0).wait()
        pltpu.make_async_copy(pbuf1, outidx_hbm.at[pl.ds(0, CE)], pssem1).wait()

    return gather_kernel


_gather = _make_gather()


def kernel(spd_index, spd_val, edge_index, spd_emb_weight):
    px = spd_index.T.reshape(CTOT, 128, 2).transpose(0, 2, 1)
    tab_r = jnp.broadcast_to(spd_emb_weight[:, :, None],
                             (IN_DIM, OUT_DIM, 16)).reshape(-1)
    v4, o4 = _gather(tab_r, spd_val, px)
    out_val = v4.transpose(1, 3, 0, 2).reshape(E, OUT_DIM)
    out_idx = o4.transpose(0, 2, 1).reshape(E, 2).T
    return (out_idx, out_val)
